# baseline (speedup 1.0000x reference)
import functools

import jax
import jax.numpy as jnp
from jax import lax
from jax.experimental import pallas as pl
from jax.experimental.pallas import tpu as pltpu
from jax.experimental.pallas import tpu_sc as plsc


def _make_sc_rope(b_tot, h, n, d):
    info = plsc.get_sparse_core_info()
    nw = info.num_cores * info.num_subcores
    tg_n = 8
    hg_n = nw // tg_n
    ntok = n // tg_n
    nh = h // hg_n
    mesh = plsc.VectorSubcoreMesh(core_axis_name="c", subcore_axis_name="s")

    npass = 2
    nhu = nh // npass
    buf = lambda: pltpu.VMEM((nhu, ntok, d), jnp.float32)
    row = lambda: pltpu.VMEM((ntok, 2 * d), jnp.float32)

    @functools.partial(
        pl.kernel, mesh=mesh,
        out_type=jax.ShapeDtypeStruct((b_tot, h, n, d), jnp.float32),
        scratch_types=[
            pltpu.VMEM((ntok,), jnp.int32), pltpu.VMEM((ntok,), jnp.int32),
            row(), row(),
            buf(), buf(), buf(), buf(),
            pltpu.SemaphoreType.DMA, pltpu.SemaphoreType.DMA,
            pltpu.SemaphoreType.DMA, pltpu.SemaphoreType.DMA,
        ],
    )
    def sc_rope(ids_hbm, comb_hbm, t_hbm, out_hbm,
                idx0, idx1, comb0, comb1,
                t0, t1, o0, o1, in_s0, in_s1, out_s0, out_s1):
        wid = lax.axis_index("s") * info.num_cores + lax.axis_index("c")
        n0 = pl.multiple_of(lax.rem(wid, tg_n) * ntok, 8)
        h0 = (wid // tg_n) * nh
        slots = ((idx0, comb0, t0, o0, in_s0, out_s0),
                 (idx1, comb1, t1, o1, in_s1, out_s1))

        def unit(u):
            return u // npass, h0 + lax.rem(u, npass) * nhu

        def start_in(u, sl):
            idx_v, comb_r, t_v, _, in_s, _ = slots[sl]
            b, ho = unit(u)
            pltpu.sync_copy(
                ids_hbm.at[pl.ds(pl.multiple_of(b * n + n0, 8), ntok)],
                idx_v)
            pltpu.make_async_copy(comb_hbm.at[idx_v], comb_r, in_s).start()
            pltpu.make_async_copy(
                t_hbm.at[b, pl.ds(ho, nhu), pl.ds(n0, ntok), :], t_v,
                in_s).start()

        def wait_in(u, sl):
            idx_v, comb_r, t_v, _, in_s, _ = slots[sl]
            b, ho = unit(u)
            pltpu.make_async_copy(comb_hbm.at[idx_v], comb_r, in_s).wait()
            pltpu.make_async_copy(
                t_hbm.at[b, pl.ds(ho, nhu), pl.ds(n0, ntok), :], t_v,
                in_s).wait()

        def out_dma(u, sl):
            _, _, _, o_v, _, out_s = slots[sl]
            b, ho = unit(u)
            return pltpu.make_async_copy(
                o_v, out_hbm.at[b, pl.ds(ho, nhu), pl.ds(n0, ntok), :],
                out_s)

        def compute(sl):
            _, comb_r, t_v, o_v, _, _ = slots[sl]
            lane = lax.iota(jnp.int32, 16)
            par = lax.rem(lane, 2)
            swp = lane + 1 - 2 * par
            sign = (2 * par - 1).astype(jnp.float32)
            dn = lax.GatherDimensionNumbers(
                offset_dims=(), collapsed_slice_dims=(0,),
                start_index_map=(0,))

            @pl.loop(0, ntok)
            def _token(nl):
                for k in range(d // 16):
                    cp = comb_r[nl, pl.ds(k * 16, 16)]
                    sp = comb_r[nl, pl.ds(d + k * 16, 16)] * sign
                    for hh in range(nhu):
                        tt = t_v[hh, nl, pl.ds(k * 16, 16)]
                        sw = lax.gather(
                            tt, swp[:, None], dn, (1,),
                            mode=lax.GatherScatterMode.PROMISE_IN_BOUNDS)
                        o_v[hh, nl, pl.ds(k * 16, 16)] = tt * cp + sw * sp

        nu = b_tot * npass
        start_in(0, 0)

        @pl.loop(0, nu, step=2)
        def _pair(u):
            @pl.when(u + 1 < nu)
            def _():
                start_in(u + 1, 1)

            wait_in(u, 0)

            @pl.when(u >= 2)
            def _():
                out_dma(u - 2, 0).wait()

            compute(0)
            out_dma(u, 0).start()

            @pl.when(u + 2 < nu)
            def _():
                start_in(u + 2, 0)

            @pl.when(u + 1 < nu)
            def _():
                wait_in(u + 1, 1)

                @pl.when(u >= 1)
                def _():
                    out_dma(u - 1, 1).wait()

                compute(1)
                out_dma(u + 1, 1).start()

        out_dma(nu - 2, 0).wait()
        out_dma(nu - 1, 1).wait()

    return sc_rope


def kernel(t, rope_ids, freqs_cos, freqs_sin):
    b, h, n, d = t.shape
    comb = jnp.concatenate([freqs_cos, freqs_sin], axis=1)
    return _make_sc_rope(b, h, n, d)(rope_ids.reshape(-1), comb, t)


# speedup vs baseline: 1.2598x; 1.2598x over previous
import jax
import jax.numpy as jnp
from jax.experimental import pallas as pl
from jax.experimental.pallas import tpu as pltpu

_HC = 8
_NBUF = 8
_OBUF = 8


def _gather_tables(ids, cos_ref, sin_ref):
    n_tok = ids.shape[0]
    n_rows, d = cos_ref.shape
    row_iota = jax.lax.broadcasted_iota(jnp.int32, (n_tok, n_rows), 1)
    onehot = (ids[:, None] == row_iota).astype(jnp.bfloat16)
    lane = jax.lax.broadcasted_iota(jnp.int32, (n_rows, d), 1)
    sin_tab = jnp.where(lane % 2 == 0, -sin_ref[...], sin_ref[...])
    cos_g = jnp.dot(onehot, cos_ref[...].astype(jnp.bfloat16),
                    preferred_element_type=jnp.float32)
    sin_g = jnp.dot(onehot, sin_tab.astype(jnp.bfloat16),
                    preferred_element_type=jnp.float32)
    return cos_g, sin_g


def _rotate_combine(tb, cos_g, sin_g):
    h, n_tok, d = tb.shape
    rowm = jax.lax.broadcasted_iota(jnp.int32, (d, d), 0)
    colm = jax.lax.broadcasted_iota(jnp.int32, (d, d), 1)
    m = ((rowm ^ 1) == colm).astype(jnp.bfloat16)
    t2 = tb.reshape(h * n_tok, d).astype(jnp.bfloat16)
    swap = jnp.dot(t2, m, preferred_element_type=jnp.float32).reshape(h, n_tok, d)
    return tb * cos_g[None] + swap * sin_g[None]


def _rope_manual(ids_ref, cos_ref, sin_ref, t_hbm, out_hbm,
                 in_buf, out_buf, in_sems, out_sems):
    b_total, h, n_tok, d = t_hbm.shape
    cpb = h // _HC
    nchunks = b_total * cpb

    def in_dma(c, slot):
        b = c // cpb
        hc = c % cpb
        return pltpu.make_async_copy(
            t_hbm.at[b, pl.ds(hc * _HC, _HC)], in_buf.at[slot],
            in_sems.at[slot])

    def out_dma(c, slot):
        b = c // cpb
        hc = c % cpb
        return pltpu.make_async_copy(
            out_buf.at[slot], out_hbm.at[b, pl.ds(hc * _HC, _HC)],
            out_sems.at[slot])

    for c in range(_NBUF):
        in_dma(c, c).start()

    def body(c, _):
        slot = jax.lax.rem(c, _NBUF)
        oslot = jax.lax.rem(c, _OBUF)
        b = c // cpb
        in_dma(c, slot).wait()
        ids = ids_ref[b, 0, :]
        cos_g, sin_g = _gather_tables(ids, cos_ref, sin_ref)
        res = _rotate_combine(in_buf[slot], cos_g, sin_g)

        @pl.when(c >= _OBUF)
        def _():
            out_dma(c - _OBUF, oslot).wait()

        out_buf[oslot] = res
        out_dma(c, oslot).start()

        @pl.when(c + _NBUF < nchunks)
        def _():
            in_dma(c + _NBUF, slot).start()

        return _

    jax.lax.fori_loop(0, nchunks, body, None)
    for k in range(_OBUF):
        c = nchunks - _OBUF + k
        out_dma(c, c % _OBUF).wait()


def kernel(t, rope_ids, freqs_cos, freqs_sin):
    b, h, n, d = t.shape
    r = freqs_cos.shape[0]
    ids3 = rope_ids.reshape(b, 1, n)
    return pl.pallas_call(
        _rope_manual,
        in_specs=[
            pl.BlockSpec(memory_space=pltpu.MemorySpace.VMEM),
            pl.BlockSpec(memory_space=pltpu.MemorySpace.VMEM),
            pl.BlockSpec(memory_space=pltpu.MemorySpace.VMEM),
            pl.BlockSpec(memory_space=pltpu.MemorySpace.HBM),
        ],
        out_specs=pl.BlockSpec(memory_space=pltpu.MemorySpace.HBM),
        out_shape=jax.ShapeDtypeStruct((b, h, n, d), t.dtype),
        scratch_shapes=[
            pltpu.VMEM((_NBUF, _HC, n, d), jnp.float32),
            pltpu.VMEM((_OBUF, _HC, n, d), jnp.float32),
            pltpu.SemaphoreType.DMA((_NBUF,)),
            pltpu.SemaphoreType.DMA((_OBUF,)),
        ],
    )(ids3, freqs_cos, freqs_sin, t)
